# baseline (speedup 1.0000x reference)
import jax
import jax.numpy as jnp
from jax import lax
from jax.experimental import pallas as pl
from jax.experimental.pallas import tpu as pltpu
from jax.experimental.pallas import tpu_sc as plsc

N = 50000
E = 800000
F_IN = 128
H = 64
HH = 32
NC = 2
NS = 16
CH = 250
OUTER = 8
EPW = E // NS
CPW = EPW // CH
NOUT = CPW // OUTER
NP = 50048
RPW = NP // NS
R = 1000
GRID = N // R

_f32 = jnp.float32
_HIGH = lax.Precision.HIGHEST


def _dot_t(a, w):
    return lax.dot_general(a, w, (((1,), (1,)), ((), ())),
                           preferred_element_type=_f32)


def _sc_mesh():
    return plsc.VectorSubcoreMesh(core_axis_name="c", subcore_axis_name="s",
                                  num_cores=NC, num_subcores=NS)


def _sc_count_body(dsts, z8, ones8, cnt_out, dst_v, ones_v, cacc):
    c = lax.axis_index("c")
    s = lax.axis_index("s")
    pltpu.sync_copy(z8.at[pl.ds(s * RPW, RPW)], cacc.at[pl.ds(s * RPW, RPW)])
    pltpu.sync_copy(ones8, ones_v)
    plsc.subcore_barrier()
    half = CPW // 2

    def outer(o, carry):
        r0 = c * half + o * OUTER
        pltpu.sync_copy(dsts.at[s, pl.ds(r0, OUTER)], dst_v)

        def inner(j, carry2):
            pltpu.sync_copy(ones_v, cacc.at[dst_v.at[j]], add=True)
            return carry2

        return lax.fori_loop(0, OUTER, inner, carry)

    lax.fori_loop(0, half // OUTER, outer, 0)
    plsc.subcore_barrier()
    pltpu.sync_copy(cacc.at[pl.ds(s * RPW, RPW)],
                    cnt_out.at[c, pl.ds(s * RPW, RPW)])


def _sc_agg_body(hcat, srcs, dsts, z32, sum_out, src_v, dst_v, rows_v, acc,
                 sems):
    c = lax.axis_index("c")
    s = lax.axis_index("s")
    pltpu.sync_copy(z32.at[pl.ds(s * RPW, RPW)], acc.at[pl.ds(s * RPW, RPW)])
    plsc.subcore_barrier()

    def outer(o, carry):
        r0 = o * OUTER
        pltpu.sync_copy(srcs.at[c, s, pl.ds(r0, OUTER)], src_v)
        pltpu.sync_copy(dsts.at[s, pl.ds(r0, OUTER)], dst_v)
        pltpu.async_copy(hcat.at[src_v.at[0]], rows_v.at[0], sems.at[0])

        def inner(j, carry2):
            p = j % 2
            pltpu.make_async_copy(hcat.at[src_v.at[j]], rows_v.at[p],
                                  sems.at[p]).wait()

            @pl.when(j + 1 < OUTER)
            def _():
                pltpu.async_copy(hcat.at[src_v.at[j + 1]], rows_v.at[1 - p],
                                 sems.at[1 - p])

            pltpu.sync_copy(rows_v.at[p], acc.at[dst_v.at[j]], add=True)
            return carry2

        return lax.fori_loop(0, OUTER, inner, carry)

    lax.fori_loop(0, NOUT, outer, 0)
    plsc.subcore_barrier()
    pltpu.sync_copy(acc.at[pl.ds(s * RPW, RPW)],
                    sum_out.at[c, pl.ds(s * RPW, RPW)])


def _sc_count(dsts, z8, ones8):
    return pl.kernel(
        _sc_count_body,
        out_type=jax.ShapeDtypeStruct((NC, NP, 8), _f32),
        mesh=_sc_mesh(),
        compiler_params=pltpu.CompilerParams(use_tc_tiling_on_sc=False),
        scratch_types=[
            pltpu.VMEM((OUTER, CH), jnp.int32),
            pltpu.VMEM((CH, 8), _f32),
            pltpu.VMEM_SHARED((NP, 8), _f32),
        ],
    )(dsts, z8, ones8)


def _sc_agg(hcat, srcs, dsts, z32):
    return pl.kernel(
        _sc_agg_body,
        out_type=jax.ShapeDtypeStruct((NC, NP, HH), _f32),
        mesh=_sc_mesh(),
        compiler_params=pltpu.CompilerParams(use_tc_tiling_on_sc=False),
        scratch_types=[
            pltpu.VMEM((OUTER, CH), jnp.int32),
            pltpu.VMEM((OUTER, CH), jnp.int32),
            pltpu.VMEM((2, CH, HH), _f32),
            pltpu.VMEM_SHARED((NP, HH), _f32),
            pltpu.SemaphoreType.DMA((2,)),
        ],
    )(hcat, srcs, dsts, z32)


def _mlp_in_body(x_ref, wp_ref, bp_ref, out_ref):
    y = _dot_t(x_ref[...], wp_ref[...])
    h = jnp.maximum(y + bp_ref[...], 0.0)
    out_ref[0] = h[:, :HH]
    out_ref[1] = h[:, HH:]


_mlp_in = pl.pallas_call(
    _mlp_in_body,
    grid=(GRID,),
    in_specs=[
        pl.BlockSpec((R, F_IN), lambda i: (i, 0)),
        pl.BlockSpec((H, F_IN), lambda i: (0, 0)),
        pl.BlockSpec((1, H), lambda i: (0, 0)),
    ],
    out_specs=pl.BlockSpec((NC, R, HH), lambda i: (0, i, 0)),
    out_shape=jax.ShapeDtypeStruct((NC, N, HH), _f32),
)


def _layer_math(sum_ref, cnt_ref, h_ref, wl_ref, bl_ref, wr_ref, g_ref,
                be_ref):
    sm = jnp.concatenate([sum_ref[0], sum_ref[1]], axis=-1)
    cnt = cnt_ref[0, :, 0:1] + cnt_ref[1, :, 0:1]
    mean = sm / jnp.maximum(cnt, 1.0)
    h = jnp.concatenate([h_ref[0], h_ref[1]], axis=-1)
    y = _dot_t(mean, wl_ref[...]) + bl_ref[...] + _dot_t(h, wr_ref[...])
    mu = jnp.mean(y, axis=-1, keepdims=True)
    var = jnp.mean((y - mu) ** 2, axis=-1, keepdims=True)
    y = (y - mu) / jnp.sqrt(var + 1e-5) * g_ref[...] + be_ref[...]
    return jnp.maximum(y, 0.0) + h


def _dense_layer_body(sum_ref, cnt_ref, h_ref, wl_ref, bl_ref, wr_ref, g_ref,
                      be_ref, out_ref):
    y = _layer_math(sum_ref, cnt_ref, h_ref, wl_ref, bl_ref, wr_ref, g_ref,
                    be_ref)
    out_ref[0] = y[:, :HH]
    out_ref[1] = y[:, HH:]


def _dense_final_body(sum_ref, cnt_ref, h_ref, wl_ref, bl_ref, wr_ref, g_ref,
                      be_ref, w1_ref, b1_ref, w2_ref, b2_ref, out_ref):
    y = _layer_math(sum_ref, cnt_ref, h_ref, wl_ref, bl_ref, wr_ref, g_ref,
                    be_ref)
    r1 = jnp.maximum(_dot_t(y, w1_ref[...]) + b1_ref[...], 0.0)
    out_ref[...] = (jnp.sum(r1 * w2_ref[...], axis=-1, keepdims=True)
                    + b2_ref[0, 0])


def _spec_half():
    return pl.BlockSpec((NC, R, HH), lambda i: (0, i, 0))


def _spec_cnt():
    return pl.BlockSpec((NC, R, 8), lambda i: (0, i, 0))


def _spec_w(shape):
    nd = len(shape)
    return pl.BlockSpec(shape, (lambda i: (0, 0)) if nd == 2 else
                        (lambda i: (0,)))


_dense_layer = pl.pallas_call(
    _dense_layer_body,
    grid=(GRID,),
    in_specs=[
        _spec_half(), _spec_cnt(), _spec_half(),
        _spec_w((H, H)), _spec_w((1, H)), _spec_w((H, H)),
        _spec_w((1, H)), _spec_w((1, H)),
    ],
    out_specs=pl.BlockSpec((NC, R, HH), lambda i: (0, i, 0)),
    out_shape=jax.ShapeDtypeStruct((NC, N, HH), _f32),
)

_dense_final = pl.pallas_call(
    _dense_final_body,
    grid=(GRID,),
    in_specs=[
        _spec_half(), _spec_cnt(), _spec_half(),
        _spec_w((H, H)), _spec_w((1, H)), _spec_w((H, H)),
        _spec_w((1, H)), _spec_w((1, H)),
        _spec_w((HH, H)), _spec_w((1, HH)), _spec_w((1, HH)),
        _spec_w((1, 1)),
    ],
    out_specs=pl.BlockSpec((R, 1), lambda i: (i, 0)),
    out_shape=jax.ShapeDtypeStruct((N, 1), _f32),
)


def kernel(x, edge_index, Wp, bp, Wl0, bl0, Wr0, g0, be0, Wl1, bl1, Wr1, g1,
           be1, W1, b1, W2, b2):
    src = edge_index[0]
    dst = edge_index[1]
    srcs = jnp.stack([src, src + N]).reshape(NC, NS, CPW, CH)
    dsts = dst.reshape(NS, CPW, CH)
    z32 = jnp.zeros((NP, HH), _f32)
    z8 = jnp.zeros((NP, 8), _f32)
    ones8 = jnp.ones((CH, 8), _f32)

    h2 = _mlp_in(x, Wp, bp.reshape(1, H))
    cnt8 = _sc_count(dsts, z8, ones8)
    sums0 = _sc_agg(h2.reshape(NC * N, HH), srcs, dsts, z32)
    h2 = _dense_layer(sums0, cnt8, h2, Wl0, bl0.reshape(1, H), Wr0,
                      g0.reshape(1, H), be0.reshape(1, H))
    sums1 = _sc_agg(h2.reshape(NC * N, HH), srcs, dsts, z32)
    out = _dense_final(sums1, cnt8, h2, Wl1, bl1.reshape(1, H), Wr1,
                       g1.reshape(1, H), be1.reshape(1, H), W1,
                       b1.reshape(1, HH), W2.reshape(1, HH),
                       b2.reshape(1, 1))
    return out[:, 0]


# speedup vs baseline: 1.0419x; 1.0419x over previous
import jax
import jax.numpy as jnp
from jax import lax
from jax.experimental import pallas as pl
from jax.experimental.pallas import tpu as pltpu
from jax.experimental.pallas import tpu_sc as plsc

N = 50000
E = 800000
F_IN = 128
H = 64
HH = 32
NC = 2
NS = 16
CH = 625
OUTER = 8
EPW = E // NS
CPW = EPW // CH
NOUT = CPW // OUTER
NP = 50048
RPW = NP // NS
R = 1000
GRID = N // R

_f32 = jnp.float32
_HIGH = lax.Precision.HIGHEST


def _dot_t(a, w):
    return lax.dot_general(a, w, (((1,), (1,)), ((), ())),
                           preferred_element_type=_f32)


def _sc_mesh():
    return plsc.VectorSubcoreMesh(core_axis_name="c", subcore_axis_name="s",
                                  num_cores=NC, num_subcores=NS)


def _sc_count_body(dsts, z8, ones8, cnt_out, dst_v, ones_v, cacc):
    c = lax.axis_index("c")
    s = lax.axis_index("s")
    pltpu.sync_copy(z8.at[pl.ds(s * RPW, RPW)], cacc.at[pl.ds(s * RPW, RPW)])
    pltpu.sync_copy(ones8, ones_v)
    plsc.subcore_barrier()
    half = CPW // 2

    def outer(o, carry):
        r0 = c * half + o * OUTER
        pltpu.sync_copy(dsts.at[s, pl.ds(r0, OUTER)], dst_v)

        def inner(j, carry2):
            pltpu.sync_copy(ones_v, cacc.at[dst_v.at[j]], add=True)
            return carry2

        return lax.fori_loop(0, OUTER, inner, carry)

    lax.fori_loop(0, half // OUTER, outer, 0)
    plsc.subcore_barrier()
    pltpu.sync_copy(cacc.at[pl.ds(s * RPW, RPW)],
                    cnt_out.at[c, pl.ds(s * RPW, RPW)])


def _sc_agg_body(hcat, srcs, dsts, z32, sum_out, src_v, dst_v, rows_v, acc,
                 sem):
    c = lax.axis_index("c")
    s = lax.axis_index("s")
    pltpu.sync_copy(z32.at[pl.ds(s * RPW, RPW)], acc.at[pl.ds(s * RPW, RPW)])
    plsc.subcore_barrier()

    def outer(o, carry):
        r0 = o * OUTER
        pltpu.sync_copy(srcs.at[c, s, pl.ds(r0, OUTER)], src_v)
        pltpu.sync_copy(dsts.at[s, pl.ds(r0, OUTER)], dst_v)

        def inner(j, carry2):
            pltpu.async_copy(hcat.at[src_v.at[j]], rows_v, sem).wait()
            pltpu.sync_copy(rows_v, acc.at[dst_v.at[j]], add=True)
            return carry2

        return lax.fori_loop(0, OUTER, inner, carry)

    lax.fori_loop(0, NOUT, outer, 0)
    plsc.subcore_barrier()
    pltpu.sync_copy(acc.at[pl.ds(s * RPW, RPW)],
                    sum_out.at[c, pl.ds(s * RPW, RPW)])


def _sc_count(dsts, z8, ones8):
    return pl.kernel(
        _sc_count_body,
        out_type=jax.ShapeDtypeStruct((NC, NP, 8), _f32),
        mesh=_sc_mesh(),
        compiler_params=pltpu.CompilerParams(use_tc_tiling_on_sc=False),
        scratch_types=[
            pltpu.VMEM((OUTER, CH), jnp.int32),
            pltpu.VMEM((CH, 8), _f32),
            pltpu.VMEM_SHARED((NP, 8), _f32),
        ],
    )(dsts, z8, ones8)


def _sc_agg(hcat, srcs, dsts, z32):
    return pl.kernel(
        _sc_agg_body,
        out_type=jax.ShapeDtypeStruct((NC, NP, HH), _f32),
        mesh=_sc_mesh(),
        compiler_params=pltpu.CompilerParams(use_tc_tiling_on_sc=False),
        scratch_types=[
            pltpu.VMEM((OUTER, CH), jnp.int32),
            pltpu.VMEM((OUTER, CH), jnp.int32),
            pltpu.VMEM((CH, HH), _f32),
            pltpu.VMEM_SHARED((NP, HH), _f32),
            pltpu.SemaphoreType.DMA,
        ],
    )(hcat, srcs, dsts, z32)


def _mlp_in_body(x_ref, wp_ref, bp_ref, out_ref):
    y = _dot_t(x_ref[...], wp_ref[...])
    h = jnp.maximum(y + bp_ref[...], 0.0)
    out_ref[0] = h[:, :HH]
    out_ref[1] = h[:, HH:]


_mlp_in = pl.pallas_call(
    _mlp_in_body,
    grid=(GRID,),
    in_specs=[
        pl.BlockSpec((R, F_IN), lambda i: (i, 0)),
        pl.BlockSpec((H, F_IN), lambda i: (0, 0)),
        pl.BlockSpec((1, H), lambda i: (0, 0)),
    ],
    out_specs=pl.BlockSpec((NC, R, HH), lambda i: (0, i, 0)),
    out_shape=jax.ShapeDtypeStruct((NC, N, HH), _f32),
)


def _layer_math(sum_ref, cnt_ref, h_ref, wl_ref, bl_ref, wr_ref, g_ref,
                be_ref):
    sm = jnp.concatenate([sum_ref[0], sum_ref[1]], axis=-1)
    cnt = cnt_ref[0, :, 0:1] + cnt_ref[1, :, 0:1]
    mean = sm / jnp.maximum(cnt, 1.0)
    h = jnp.concatenate([h_ref[0], h_ref[1]], axis=-1)
    y = _dot_t(mean, wl_ref[...]) + bl_ref[...] + _dot_t(h, wr_ref[...])
    mu = jnp.mean(y, axis=-1, keepdims=True)
    var = jnp.mean((y - mu) ** 2, axis=-1, keepdims=True)
    y = (y - mu) / jnp.sqrt(var + 1e-5) * g_ref[...] + be_ref[...]
    return jnp.maximum(y, 0.0) + h


def _dense_layer_body(sum_ref, cnt_ref, h_ref, wl_ref, bl_ref, wr_ref, g_ref,
                      be_ref, out_ref):
    y = _layer_math(sum_ref, cnt_ref, h_ref, wl_ref, bl_ref, wr_ref, g_ref,
                    be_ref)
    out_ref[0] = y[:, :HH]
    out_ref[1] = y[:, HH:]


def _dense_final_body(sum_ref, cnt_ref, h_ref, wl_ref, bl_ref, wr_ref, g_ref,
                      be_ref, w1_ref, b1_ref, w2_ref, b2_ref, out_ref):
    y = _layer_math(sum_ref, cnt_ref, h_ref, wl_ref, bl_ref, wr_ref, g_ref,
                    be_ref)
    r1 = jnp.maximum(_dot_t(y, w1_ref[...]) + b1_ref[...], 0.0)
    out_ref[...] = (jnp.sum(r1 * w2_ref[...], axis=-1, keepdims=True)
                    + b2_ref[0, 0])


def _spec_half():
    return pl.BlockSpec((NC, R, HH), lambda i: (0, i, 0))


def _spec_cnt():
    return pl.BlockSpec((NC, R, 8), lambda i: (0, i, 0))


def _spec_w(shape):
    nd = len(shape)
    return pl.BlockSpec(shape, (lambda i: (0, 0)) if nd == 2 else
                        (lambda i: (0,)))


_dense_layer = pl.pallas_call(
    _dense_layer_body,
    grid=(GRID,),
    in_specs=[
        _spec_half(), _spec_cnt(), _spec_half(),
        _spec_w((H, H)), _spec_w((1, H)), _spec_w((H, H)),
        _spec_w((1, H)), _spec_w((1, H)),
    ],
    out_specs=pl.BlockSpec((NC, R, HH), lambda i: (0, i, 0)),
    out_shape=jax.ShapeDtypeStruct((NC, N, HH), _f32),
)

_dense_final = pl.pallas_call(
    _dense_final_body,
    grid=(GRID,),
    in_specs=[
        _spec_half(), _spec_cnt(), _spec_half(),
        _spec_w((H, H)), _spec_w((1, H)), _spec_w((H, H)),
        _spec_w((1, H)), _spec_w((1, H)),
        _spec_w((HH, H)), _spec_w((1, HH)), _spec_w((1, HH)),
        _spec_w((1, 1)),
    ],
    out_specs=pl.BlockSpec((R, 1), lambda i: (i, 0)),
    out_shape=jax.ShapeDtypeStruct((N, 1), _f32),
)


def kernel(x, edge_index, Wp, bp, Wl0, bl0, Wr0, g0, be0, Wl1, bl1, Wr1, g1,
           be1, W1, b1, W2, b2):
    src = edge_index[0]
    dst = edge_index[1]
    srcs = jnp.stack([src, src + N]).reshape(NC, NS, CPW, CH)
    dsts = dst.reshape(NS, CPW, CH)
    z32 = jnp.zeros((NP, HH), _f32)
    z8 = jnp.zeros((NP, 8), _f32)
    ones8 = jnp.ones((CH, 8), _f32)

    h2 = _mlp_in(x, Wp, bp.reshape(1, H))
    cnt8 = _sc_count(dsts, z8, ones8)
    sums0 = _sc_agg(h2.reshape(NC * N, HH), srcs, dsts, z32)
    h2 = _dense_layer(sums0, cnt8, h2, Wl0, bl0.reshape(1, H), Wr0,
                      g0.reshape(1, H), be0.reshape(1, H))
    sums1 = _sc_agg(h2.reshape(NC * N, HH), srcs, dsts, z32)
    out = _dense_final(sums1, cnt8, h2, Wl1, bl1.reshape(1, H), Wr1,
                       g1.reshape(1, H), be1.reshape(1, H), W1,
                       b1.reshape(1, HH), W2.reshape(1, HH),
                       b2.reshape(1, 1))
    return out[:, 0]
